# Initial kernel scaffold; baseline (speedup 1.0000x reference)
import jax
import jax.numpy as jnp
from jax.experimental import pallas as pl


def kernel(x, coords, w1, w2, bn1_w, bn1_b, bn2_w, bn2_b):
    raise NotImplementedError("write your pallas kernel here")


# speedup vs baseline: 11.5054x; 11.5054x over previous
import functools

import jax
import jax.numpy as jnp
from jax import lax
from jax.experimental import pallas as pl
from jax.experimental.pallas import tpu as pltpu
from jax.experimental.pallas import tpu_sc as plsc

B, N, CH = 16, 4096, 32
S, K = 512, 32
C1, C2 = 64, 64
RADIUS2 = 0.25
MCOUNT = float(B * S * K)
EPS = 1e-5


def _fps_body(cx_ref, cy_ref, cz_ref, nxyz_ref, idx_ref, dist_ref):
    dist_ref[...] = jnp.full((B, N), 1e10, dtype=jnp.float32)
    iota = lax.broadcasted_iota(jnp.int32, (B, N), 1)
    cxv = cx_ref[...]
    cyv = cy_ref[...]
    czv = cz_ref[...]

    def step(t, lastv):
        mask = iota == lastv
        ccx = jnp.sum(jnp.where(mask, cxv, 0.0), axis=1, keepdims=True)
        ccy = jnp.sum(jnp.where(mask, cyv, 0.0), axis=1, keepdims=True)
        ccz = jnp.sum(jnp.where(mask, czv, 0.0), axis=1, keepdims=True)
        nxyz_ref[pl.ds(t, 1), :, :] = jnp.concatenate(
            [ccx, ccy, ccz], axis=1).reshape(1, B, 3)
        idx_ref[pl.ds(t, 1), :, :] = lastv.reshape(1, B, 1)
        d = (cxv - ccx) ** 2 + (cyv - ccy) ** 2 + (czv - ccz) ** 2
        dd = jnp.minimum(dist_ref[...], d)
        dist_ref[...] = dd
        m = jnp.max(dd, axis=1, keepdims=True)
        cand = jnp.where(dd == m, iota, N)
        return jnp.min(cand, axis=1, keepdims=True).astype(jnp.int32)

    lax.fori_loop(0, S, step, jnp.zeros((B, 1), jnp.int32))


def _run_fps(cx, cy, cz):
    return pl.pallas_call(
        _fps_body,
        out_shape=(jax.ShapeDtypeStruct((S, B, 3), jnp.float32),
                   jax.ShapeDtypeStruct((S, B, 1), jnp.int32)),
        scratch_shapes=[pltpu.VMEM((B, N), jnp.float32)],
    )(cx, cy, cz)


S_TILE = 128


def _bq_body(nx_ref, ct_ref, out_ref):
    cm = nx_ref[0]
    p3 = ct_ref[0]
    dot = lax.dot_general(cm.astype(jnp.bfloat16), p3.astype(jnp.bfloat16),
                          (((1,), (0,)), ((), ())),
                          preferred_element_type=jnp.float32)
    cc = jnp.sum(cm * cm, axis=1, keepdims=True)
    pp = jnp.sum(p3 * p3, axis=0, keepdims=True)
    sqd = (-2.0 * dot + cc) + pp
    iota = lax.broadcasted_iota(jnp.int32, (S_TILE, N), 1)
    cand = jnp.where(sqd <= RADIUS2, iota, N)
    cols = []
    for k in range(K):
        first = jnp.min(cand, axis=1, keepdims=True)
        cols.append(first)
        if k + 1 < K:
            cand = jnp.where(cand == first, N, cand)
    idxm = jnp.concatenate(cols, axis=1)
    idxm = jnp.where(idxm == N, cols[0], idxm)
    out_ref[0] = idxm


def _run_ballquery(new_xyz, ct):
    return pl.pallas_call(
        _bq_body,
        grid=(B, S // S_TILE),
        in_specs=[
            pl.BlockSpec((1, S_TILE, 3), lambda b, s: (b, s, 0)),
            pl.BlockSpec((1, 3, N), lambda b, s: (b, 0, 0)),
        ],
        out_specs=pl.BlockSpec((1, S_TILE, K), lambda b, s: (b, s, 0)),
        out_shape=jax.ShapeDtypeStruct((B, S, K), jnp.int32),
    )(new_xyz, ct)


TOTAL_ROWS = B * S * K + B * S
GCHUNK = 128


def _sc_gather(gidx2d, table):
    info = plsc.get_sparse_core_info()
    nw = info.num_cores * info.num_subcores
    rows_per_w = TOTAL_ROWS // nw
    chunks = rows_per_w // GCHUNK
    idx_rows = rows_per_w // GCHUNK
    mesh = plsc.VectorSubcoreMesh(core_axis_name="c", subcore_axis_name="s")

    @functools.partial(
        pl.kernel, mesh=mesh,
        compiler_params=pltpu.CompilerParams(use_tc_tiling_on_sc=False),
        out_type=jax.ShapeDtypeStruct((TOTAL_ROWS, CH), jnp.float32),
        scratch_types=[
            pltpu.VMEM((idx_rows, GCHUNK), jnp.int32),
            pltpu.VMEM((GCHUNK, CH), jnp.float32),
            pltpu.SemaphoreType.DMA,
        ],
    )
    def gk(idx_hbm, table_hbm, out_hbm, idx_v, rows_v, sem):
        wid = lax.axis_index("s") * info.num_cores + lax.axis_index("c")
        pltpu.sync_copy(idx_hbm.at[pl.ds(wid * idx_rows, idx_rows)], idx_v)
        base = wid * rows_per_w

        def chunk(c, carry):
            pltpu.async_copy(table_hbm.at[idx_v.at[c]], rows_v, sem).wait()
            pltpu.sync_copy(rows_v, out_hbm.at[pl.ds(base + c * GCHUNK, GCHUNK)])
            return carry

        lax.fori_loop(0, chunks, chunk, 0)

    return gk(gidx2d, table)


G_TILE = 64
GROWS = G_TILE * K
N_TILES = (B * S) // G_TILE


def _build_f(g_ref, np_ref):
    g3 = g_ref[...].reshape(G_TILE, K, CH)
    npb = np_ref[...]
    np3 = jnp.broadcast_to(npb[:, None, :], (G_TILE, K, CH))
    f3 = jnp.concatenate([g3 - np3, np3], axis=2)
    return f3.reshape(GROWS, 2 * CH)


_DN_RC = (((0,), (0,)), ((), ()))
_DN_CC = (((1,), (1,)), ((), ()))


def _eye64():
    r = lax.broadcasted_iota(jnp.int32, (C1, C1), 0)
    c = lax.broadcasted_iota(jnp.int32, (C1, C1), 1)
    return (r == c).astype(jnp.float32)


def _bn_scale_shift(W, m_mat, s_vec, gw, gb):
    mean_y = lax.dot_general(s_vec / MCOUNT, W, _DN_CC,
                             preferred_element_type=jnp.float32)
    A = lax.dot_general(W, m_mat, (((1,), (0,)), ((), ())),
                        preferred_element_type=jnp.float32)
    Km = lax.dot_general(A, W, _DN_CC, preferred_element_type=jnp.float32)
    e2 = jnp.sum(Km * _eye64(), axis=0, keepdims=True) / MCOUNT
    var = e2 - mean_y * mean_y
    scale = gw * lax.rsqrt(var + EPS)
    shift = gb - mean_y * scale
    return scale, shift


def _passA_body(g_ref, np_ref, m1_ref, s1_ref):
    f = _build_f(g_ref, np_ref)
    m1 = lax.dot_general(f, f, _DN_RC, preferred_element_type=jnp.float32)
    s1 = jnp.sum(f, axis=0, keepdims=True)

    @pl.when(pl.program_id(0) == 0)
    def _():
        m1_ref[...] = jnp.zeros_like(m1_ref)
        s1_ref[...] = jnp.zeros_like(s1_ref)

    m1_ref[...] += m1
    s1_ref[...] += s1


def _passB_body(g_ref, np_ref, w1_ref, m1_ref, s1_ref, g1w_ref, g1b_ref,
                m2_ref, s2_ref):
    f = _build_f(g_ref, np_ref)
    W1 = w1_ref[...]
    scale1, shift1 = _bn_scale_shift(W1, m1_ref[...], s1_ref[...],
                                     g1w_ref[...], g1b_ref[...])
    y1 = lax.dot_general(f, W1, _DN_CC, preferred_element_type=jnp.float32)
    z1 = jnp.maximum(y1 * scale1 + shift1, 0.0)
    m2 = lax.dot_general(z1, z1, _DN_RC, preferred_element_type=jnp.float32)
    s2 = jnp.sum(z1, axis=0, keepdims=True)

    @pl.when(pl.program_id(0) == 0)
    def _():
        m2_ref[...] = jnp.zeros_like(m2_ref)
        s2_ref[...] = jnp.zeros_like(s2_ref)

    m2_ref[...] += m2
    s2_ref[...] += s2


def _passC_body(g_ref, np_ref, w1_ref, m1_ref, s1_ref, g1w_ref, g1b_ref,
                w2_ref, m2_ref, s2_ref, g2w_ref, g2b_ref, out_ref):
    f = _build_f(g_ref, np_ref)
    W1 = w1_ref[...]
    scale1, shift1 = _bn_scale_shift(W1, m1_ref[...], s1_ref[...],
                                     g1w_ref[...], g1b_ref[...])
    y1 = lax.dot_general(f, W1, _DN_CC, preferred_element_type=jnp.float32)
    z1 = jnp.maximum(y1 * scale1 + shift1, 0.0)
    W2 = w2_ref[...]
    scale2, shift2 = _bn_scale_shift(W2, m2_ref[...], s2_ref[...],
                                     g2w_ref[...], g2b_ref[...])
    y2 = lax.dot_general(z1, W2, _DN_CC, preferred_element_type=jnp.float32)
    z2 = jnp.maximum(y2 * scale2 + shift2, 0.0)
    out_ref[...] = jnp.max(z2.reshape(G_TILE, K, C2), axis=1)


_SMALL = lambda shp: pl.BlockSpec(shp, lambda i: tuple(0 for _ in shp))


def _run_mlp(G, NP2, W1, W2, g1w, g1b, g2w, g2b):
    g_spec = pl.BlockSpec((GROWS, CH), lambda i: (i, 0))
    np_spec = pl.BlockSpec((G_TILE, CH), lambda i: (i, 0))
    acc_specs = (pl.BlockSpec((C1, C1), lambda i: (0, 0)),
                 pl.BlockSpec((1, C1), lambda i: (0, 0)))
    acc_shapes = (jax.ShapeDtypeStruct((C1, C1), jnp.float32),
                  jax.ShapeDtypeStruct((1, C1), jnp.float32))

    M1, s1 = pl.pallas_call(
        _passA_body, grid=(N_TILES,),
        in_specs=[g_spec, np_spec],
        out_specs=acc_specs, out_shape=acc_shapes,
    )(G, NP2)

    M2, s2 = pl.pallas_call(
        _passB_body, grid=(N_TILES,),
        in_specs=[g_spec, np_spec, _SMALL((C1, C1)), _SMALL((C1, C1)),
                  _SMALL((1, C1)), _SMALL((1, C1)), _SMALL((1, C1))],
        out_specs=acc_specs, out_shape=acc_shapes,
    )(G, NP2, W1, M1, s1, g1w, g1b)

    out2d = pl.pallas_call(
        _passC_body, grid=(N_TILES,),
        in_specs=[g_spec, np_spec, _SMALL((C1, C1)), _SMALL((C1, C1)),
                  _SMALL((1, C1)), _SMALL((1, C1)), _SMALL((1, C1)),
                  _SMALL((C2, C2)), _SMALL((C2, C2)), _SMALL((1, C2)),
                  _SMALL((1, C2)), _SMALL((1, C2))],
        out_specs=pl.BlockSpec((G_TILE, C2), lambda i: (i, 0)),
        out_shape=jax.ShapeDtypeStruct((B * S, C2), jnp.float32),
    )(G, NP2, W1, M1, s1, g1w, g1b, W2, M2, s2, g2w, g2b)
    return out2d


def kernel(x, coords, w1, w2, bn1_w, bn1_b, bn2_w, bn2_b):
    cx = coords[:, :, 0]
    cy = coords[:, :, 1]
    cz = coords[:, :, 2]
    ct = jnp.transpose(coords, (0, 2, 1))

    nxyz_tm, fidx_tm = _run_fps(cx, cy, cz)
    new_xyz = jnp.transpose(nxyz_tm, (1, 0, 2))
    fps_idx = jnp.transpose(fidx_tm[:, :, 0], (1, 0))
    idx = _run_ballquery(new_xyz, ct)

    boff = (jnp.arange(B, dtype=jnp.int32) * N)
    gidx = (idx + boff[:, None, None]).reshape(-1)
    cgidx = (fps_idx + boff[:, None]).reshape(-1)
    gidx2d = jnp.concatenate([gidx, cgidx]).reshape(TOTAL_ROWS // GCHUNK, GCHUNK)
    table = jnp.transpose(x, (0, 2, 1)).reshape(B * N, CH)
    Gall = _sc_gather(gidx2d, table)
    G = Gall[:B * S * K]
    NP2 = Gall[B * S * K:]

    out2d = _run_mlp(G, NP2, w1[:, :, 0], w2[:, :, 0],
                     bn1_w.reshape(1, C1), bn1_b.reshape(1, C1),
                     bn2_w.reshape(1, C2), bn2_b.reshape(1, C2))

    y = jnp.transpose(out2d.reshape(B, S, C2), (0, 2, 1))
    return (new_xyz, y)


# speedup vs baseline: 13.1336x; 1.1415x over previous
import functools

import jax
import jax.numpy as jnp
from jax import lax
from jax.experimental import pallas as pl
from jax.experimental.pallas import tpu as pltpu
from jax.experimental.pallas import tpu_sc as plsc

B, N, CH = 16, 4096, 32
S, K = 512, 32
C1, C2 = 64, 64
RADIUS2 = 0.25
MCOUNT = float(B * S * K)
EPS = 1e-5


def _fps_body(cx_ref, cy_ref, cz_ref, nxyz_ref, idx_ref, dist_ref):
    dist_ref[...] = jnp.full((B, N), 1e10, dtype=jnp.float32)
    iota = lax.broadcasted_iota(jnp.int32, (B, N), 1)
    cxv = cx_ref[...]
    cyv = cy_ref[...]
    czv = cz_ref[...]

    def step(t, lastv):
        mask = iota == lastv
        ccx = jnp.sum(jnp.where(mask, cxv, 0.0), axis=1, keepdims=True)
        ccy = jnp.sum(jnp.where(mask, cyv, 0.0), axis=1, keepdims=True)
        ccz = jnp.sum(jnp.where(mask, czv, 0.0), axis=1, keepdims=True)
        nxyz_ref[pl.ds(t, 1), :, :] = jnp.concatenate(
            [ccx, ccy, ccz], axis=1).reshape(1, B, 3)
        idx_ref[pl.ds(t, 1), :, :] = lastv.reshape(1, B, 1)
        d = (cxv - ccx) ** 2 + (cyv - ccy) ** 2 + (czv - ccz) ** 2
        dd = jnp.minimum(dist_ref[...], d)
        dist_ref[...] = dd
        m = jnp.max(dd, axis=1, keepdims=True)
        cand = jnp.where(dd == m, iota, N)
        return jnp.min(cand, axis=1, keepdims=True).astype(jnp.int32)

    lax.fori_loop(0, S, step, jnp.zeros((B, 1), jnp.int32))


def _run_fps(cx, cy, cz):
    return pl.pallas_call(
        _fps_body,
        out_shape=(jax.ShapeDtypeStruct((S, B, 3), jnp.float32),
                   jax.ShapeDtypeStruct((S, B, 1), jnp.int32)),
        scratch_shapes=[pltpu.VMEM((B, N), jnp.float32)],
    )(cx, cy, cz)


S_TILE = 128


BQ_CHUNK = 512
BIG = 1 << 20


def _bq_body(nx_ref, ct_ref, out_ref):
    cm = nx_ref[0]
    cmb = cm.astype(jnp.bfloat16)
    cc = jnp.sum(cm * cm, axis=1, keepdims=True)
    lanes = lax.broadcasted_iota(jnp.int32, (S_TILE, K), 1)
    citer = lax.broadcasted_iota(jnp.int32, (S_TILE, BQ_CHUNK), 1)

    def do_chunk(c, idxm, count):
        p3 = ct_ref[0, :, pl.ds(c * BQ_CHUNK, BQ_CHUNK)]
        dot = lax.dot_general(cmb, p3.astype(jnp.bfloat16),
                              (((1,), (0,)), ((), ())),
                              preferred_element_type=jnp.float32)
        pp = jnp.sum(p3 * p3, axis=0, keepdims=True)
        sqd = (-2.0 * dot + cc) + pp
        cand0 = jnp.where(sqd <= RADIUS2, citer + c * BQ_CHUNK, BIG)

        def extract(_, carry):
            idxm, count, cand = carry
            first = jnp.min(cand, axis=1, keepdims=True)
            hit = first < BIG
            idxm = jnp.where((lanes == count) & hit, first, idxm)
            count = count + hit.astype(jnp.int32)
            cand = jnp.where(cand == first, BIG, cand)
            return idxm, count, cand

        idxm, count, _ = lax.fori_loop(0, K, extract, (idxm, count, cand0))
        return idxm, count

    def chunk_step(c, carry):
        idxm, count = carry
        return lax.cond(jnp.min(count) < K,
                        lambda a, b: do_chunk(c, a, b),
                        lambda a, b: (a, b), idxm, count)

    idxm0 = jnp.full((S_TILE, K), N, jnp.int32)
    count0 = jnp.zeros((S_TILE, 1), jnp.int32)
    idxm, _ = lax.fori_loop(0, N // BQ_CHUNK, chunk_step, (idxm0, count0))
    idxm = jnp.where(idxm == N, idxm[:, 0:1], idxm)
    out_ref[0] = idxm


def _run_ballquery(new_xyz, ct):
    return pl.pallas_call(
        _bq_body,
        grid=(B, S // S_TILE),
        in_specs=[
            pl.BlockSpec((1, S_TILE, 3), lambda b, s: (b, s, 0)),
            pl.BlockSpec((1, 3, N), lambda b, s: (b, 0, 0)),
        ],
        out_specs=pl.BlockSpec((1, S_TILE, K), lambda b, s: (b, s, 0)),
        out_shape=jax.ShapeDtypeStruct((B, S, K), jnp.int32),
    )(new_xyz, ct)


TOTAL_ROWS = B * S * K + B * S
GCHUNK = 128


def _sc_gather(gidx2d, table):
    info = plsc.get_sparse_core_info()
    nw = info.num_cores * info.num_subcores
    rows_per_w = TOTAL_ROWS // nw
    chunks = rows_per_w // GCHUNK
    idx_rows = rows_per_w // GCHUNK
    mesh = plsc.VectorSubcoreMesh(core_axis_name="c", subcore_axis_name="s")

    @functools.partial(
        pl.kernel, mesh=mesh,
        compiler_params=pltpu.CompilerParams(use_tc_tiling_on_sc=False),
        out_type=jax.ShapeDtypeStruct((TOTAL_ROWS, CH), jnp.float32),
        scratch_types=[
            pltpu.VMEM((idx_rows, GCHUNK), jnp.int32),
            pltpu.VMEM((GCHUNK, CH), jnp.float32),
            pltpu.SemaphoreType.DMA,
        ],
    )
    def gk(idx_hbm, table_hbm, out_hbm, idx_v, rows_v, sem):
        wid = lax.axis_index("s") * info.num_cores + lax.axis_index("c")
        pltpu.sync_copy(idx_hbm.at[pl.ds(wid * idx_rows, idx_rows)], idx_v)
        base = wid * rows_per_w

        def chunk(c, carry):
            pltpu.async_copy(table_hbm.at[idx_v.at[c]], rows_v, sem).wait()
            pltpu.sync_copy(rows_v, out_hbm.at[pl.ds(base + c * GCHUNK, GCHUNK)])
            return carry

        lax.fori_loop(0, chunks, chunk, 0)

    return gk(gidx2d, table)


G_TILE = 64
GROWS = G_TILE * K
N_TILES = (B * S) // G_TILE


def _build_f(g_ref, np_ref):
    g3 = g_ref[...].reshape(G_TILE, K, CH)
    npb = np_ref[...]
    np3 = jnp.broadcast_to(npb[:, None, :], (G_TILE, K, CH))
    f3 = jnp.concatenate([g3 - np3, np3], axis=2)
    return f3.reshape(GROWS, 2 * CH)


_DN_RC = (((0,), (0,)), ((), ()))
_DN_CC = (((1,), (1,)), ((), ()))


def _eye64():
    r = lax.broadcasted_iota(jnp.int32, (C1, C1), 0)
    c = lax.broadcasted_iota(jnp.int32, (C1, C1), 1)
    return (r == c).astype(jnp.float32)


def _bn_scale_shift(W, m_mat, s_vec, gw, gb):
    mean_y = lax.dot_general(s_vec / MCOUNT, W, _DN_CC,
                             preferred_element_type=jnp.float32)
    A = lax.dot_general(W, m_mat, (((1,), (0,)), ((), ())),
                        preferred_element_type=jnp.float32)
    Km = lax.dot_general(A, W, _DN_CC, preferred_element_type=jnp.float32)
    e2 = jnp.sum(Km * _eye64(), axis=0, keepdims=True) / MCOUNT
    var = e2 - mean_y * mean_y
    scale = gw * lax.rsqrt(var + EPS)
    shift = gb - mean_y * scale
    return scale, shift


def _passA_body(g_ref, np_ref, m1_ref, s1_ref):
    f = _build_f(g_ref, np_ref)
    m1 = lax.dot_general(f, f, _DN_RC, preferred_element_type=jnp.float32)
    s1 = jnp.sum(f, axis=0, keepdims=True)

    @pl.when(pl.program_id(0) == 0)
    def _():
        m1_ref[...] = jnp.zeros_like(m1_ref)
        s1_ref[...] = jnp.zeros_like(s1_ref)

    m1_ref[...] += m1
    s1_ref[...] += s1


def _passB_body(g_ref, np_ref, w1_ref, m1_ref, s1_ref, g1w_ref, g1b_ref,
                m2_ref, s2_ref):
    f = _build_f(g_ref, np_ref)
    W1 = w1_ref[...]
    scale1, shift1 = _bn_scale_shift(W1, m1_ref[...], s1_ref[...],
                                     g1w_ref[...], g1b_ref[...])
    y1 = lax.dot_general(f, W1, _DN_CC, preferred_element_type=jnp.float32)
    z1 = jnp.maximum(y1 * scale1 + shift1, 0.0)
    m2 = lax.dot_general(z1, z1, _DN_RC, preferred_element_type=jnp.float32)
    s2 = jnp.sum(z1, axis=0, keepdims=True)

    @pl.when(pl.program_id(0) == 0)
    def _():
        m2_ref[...] = jnp.zeros_like(m2_ref)
        s2_ref[...] = jnp.zeros_like(s2_ref)

    m2_ref[...] += m2
    s2_ref[...] += s2


def _passC_body(g_ref, np_ref, w1_ref, m1_ref, s1_ref, g1w_ref, g1b_ref,
                w2_ref, m2_ref, s2_ref, g2w_ref, g2b_ref, out_ref):
    f = _build_f(g_ref, np_ref)
    W1 = w1_ref[...]
    scale1, shift1 = _bn_scale_shift(W1, m1_ref[...], s1_ref[...],
                                     g1w_ref[...], g1b_ref[...])
    y1 = lax.dot_general(f, W1, _DN_CC, preferred_element_type=jnp.float32)
    z1 = jnp.maximum(y1 * scale1 + shift1, 0.0)
    W2 = w2_ref[...]
    scale2, shift2 = _bn_scale_shift(W2, m2_ref[...], s2_ref[...],
                                     g2w_ref[...], g2b_ref[...])
    y2 = lax.dot_general(z1, W2, _DN_CC, preferred_element_type=jnp.float32)
    z2 = jnp.maximum(y2 * scale2 + shift2, 0.0)
    out_ref[...] = jnp.max(z2.reshape(G_TILE, K, C2), axis=1)


_SMALL = lambda shp: pl.BlockSpec(shp, lambda i: tuple(0 for _ in shp))


def _run_mlp(G, NP2, W1, W2, g1w, g1b, g2w, g2b):
    g_spec = pl.BlockSpec((GROWS, CH), lambda i: (i, 0))
    np_spec = pl.BlockSpec((G_TILE, CH), lambda i: (i, 0))
    acc_specs = (pl.BlockSpec((C1, C1), lambda i: (0, 0)),
                 pl.BlockSpec((1, C1), lambda i: (0, 0)))
    acc_shapes = (jax.ShapeDtypeStruct((C1, C1), jnp.float32),
                  jax.ShapeDtypeStruct((1, C1), jnp.float32))

    M1, s1 = pl.pallas_call(
        _passA_body, grid=(N_TILES,),
        in_specs=[g_spec, np_spec],
        out_specs=acc_specs, out_shape=acc_shapes,
    )(G, NP2)

    M2, s2 = pl.pallas_call(
        _passB_body, grid=(N_TILES,),
        in_specs=[g_spec, np_spec, _SMALL((C1, C1)), _SMALL((C1, C1)),
                  _SMALL((1, C1)), _SMALL((1, C1)), _SMALL((1, C1))],
        out_specs=acc_specs, out_shape=acc_shapes,
    )(G, NP2, W1, M1, s1, g1w, g1b)

    out2d = pl.pallas_call(
        _passC_body, grid=(N_TILES,),
        in_specs=[g_spec, np_spec, _SMALL((C1, C1)), _SMALL((C1, C1)),
                  _SMALL((1, C1)), _SMALL((1, C1)), _SMALL((1, C1)),
                  _SMALL((C2, C2)), _SMALL((C2, C2)), _SMALL((1, C2)),
                  _SMALL((1, C2)), _SMALL((1, C2))],
        out_specs=pl.BlockSpec((G_TILE, C2), lambda i: (i, 0)),
        out_shape=jax.ShapeDtypeStruct((B * S, C2), jnp.float32),
    )(G, NP2, W1, M1, s1, g1w, g1b, W2, M2, s2, g2w, g2b)
    return out2d


def kernel(x, coords, w1, w2, bn1_w, bn1_b, bn2_w, bn2_b):
    cx = coords[:, :, 0]
    cy = coords[:, :, 1]
    cz = coords[:, :, 2]
    ct = jnp.transpose(coords, (0, 2, 1))

    nxyz_tm, fidx_tm = _run_fps(cx, cy, cz)
    new_xyz = jnp.transpose(nxyz_tm, (1, 0, 2))
    fps_idx = jnp.transpose(fidx_tm[:, :, 0], (1, 0))
    idx = _run_ballquery(new_xyz, ct)

    boff = (jnp.arange(B, dtype=jnp.int32) * N)
    gidx = (idx + boff[:, None, None]).reshape(-1)
    cgidx = (fps_idx + boff[:, None]).reshape(-1)
    gidx2d = jnp.concatenate([gidx, cgidx]).reshape(TOTAL_ROWS // GCHUNK, GCHUNK)
    table = jnp.transpose(x, (0, 2, 1)).reshape(B * N, CH)
    Gall = _sc_gather(gidx2d, table)
    G = Gall[:B * S * K]
    NP2 = Gall[B * S * K:]

    out2d = _run_mlp(G, NP2, w1[:, :, 0], w2[:, :, 0],
                     bn1_w.reshape(1, C1), bn1_b.reshape(1, C1),
                     bn2_w.reshape(1, C2), bn2_b.reshape(1, C2))

    y = jnp.transpose(out2d.reshape(B, S, C2), (0, 2, 1))
    return (new_xyz, y)


# speedup vs baseline: 14.7589x; 1.1237x over previous
import functools

import jax
import jax.numpy as jnp
from jax import lax
from jax.experimental import pallas as pl
from jax.experimental.pallas import tpu as pltpu
from jax.experimental.pallas import tpu_sc as plsc

B, N, CH = 16, 4096, 32
S, K = 512, 32
C1, C2 = 64, 64
RADIUS2 = 0.25
MCOUNT = float(B * S * K)
EPS = 1e-5


def _fps_body(cx_ref, cy_ref, cz_ref, nxyz_ref, idx_ref, dist_ref):
    dist_ref[...] = jnp.full((B, N), 1e10, dtype=jnp.float32)
    iota = lax.broadcasted_iota(jnp.int32, (B, N), 1)
    cxv = cx_ref[...]
    cyv = cy_ref[...]
    czv = cz_ref[...]

    def step(t, lastv):
        mask = iota == lastv
        ccx = jnp.sum(jnp.where(mask, cxv, 0.0), axis=1, keepdims=True)
        ccy = jnp.sum(jnp.where(mask, cyv, 0.0), axis=1, keepdims=True)
        ccz = jnp.sum(jnp.where(mask, czv, 0.0), axis=1, keepdims=True)
        nxyz_ref[pl.ds(t, 1), :, :] = jnp.concatenate(
            [ccx, ccy, ccz], axis=1).reshape(1, B, 3)
        idx_ref[pl.ds(t, 1), :, :] = lastv.reshape(1, B, 1)
        d = (cxv - ccx) ** 2 + (cyv - ccy) ** 2 + (czv - ccz) ** 2
        dd = jnp.minimum(dist_ref[...], d)
        dist_ref[...] = dd
        m = jnp.max(dd, axis=1, keepdims=True)
        cand = jnp.where(dd == m, iota, N)
        return jnp.min(cand, axis=1, keepdims=True).astype(jnp.int32)

    lax.fori_loop(0, S, step, jnp.zeros((B, 1), jnp.int32))


def _run_fps(cx, cy, cz):
    return pl.pallas_call(
        _fps_body,
        out_shape=(jax.ShapeDtypeStruct((S, B, 3), jnp.float32),
                   jax.ShapeDtypeStruct((S, B, 1), jnp.int32)),
        scratch_shapes=[pltpu.VMEM((B, N), jnp.float32)],
    )(cx, cy, cz)


S_TILE = 512


BQ_CHUNK = 512
BIG = 1 << 20


def _bq_body(nx_ref, ct_ref, out_ref):
    cm = nx_ref[0]
    cmb = cm.astype(jnp.bfloat16)
    cc = jnp.sum(cm * cm, axis=1, keepdims=True)
    lanes = lax.broadcasted_iota(jnp.int32, (S_TILE, K), 1)
    citer = lax.broadcasted_iota(jnp.int32, (S_TILE, BQ_CHUNK), 1)

    def do_chunk(c, idxm, count):
        p3 = ct_ref[0, :, pl.ds(c * BQ_CHUNK, BQ_CHUNK)]
        dot = lax.dot_general(cmb, p3.astype(jnp.bfloat16),
                              (((1,), (0,)), ((), ())),
                              preferred_element_type=jnp.float32)
        pp = jnp.sum(p3 * p3, axis=0, keepdims=True)
        sqd = (-2.0 * dot + cc) + pp
        cand0 = jnp.where(sqd <= RADIUS2, citer + c * BQ_CHUNK, BIG)

        def extract(_, carry):
            idxm, count, cand = carry
            first = jnp.min(cand, axis=1, keepdims=True)
            hit = first < BIG
            idxm = jnp.where((lanes == count) & hit, first, idxm)
            count = count + hit.astype(jnp.int32)
            cand = jnp.where(cand == first, BIG, cand)
            return idxm, count, cand

        idxm, count, _ = lax.fori_loop(0, K, extract, (idxm, count, cand0))
        return idxm, count

    def chunk_step(c, carry):
        idxm, count = carry
        return lax.cond(jnp.min(count) < K,
                        lambda a, b: do_chunk(c, a, b),
                        lambda a, b: (a, b), idxm, count)

    idxm0 = jnp.full((S_TILE, K), N, jnp.int32)
    count0 = jnp.zeros((S_TILE, 1), jnp.int32)
    idxm, _ = lax.fori_loop(0, N // BQ_CHUNK, chunk_step, (idxm0, count0))
    idxm = jnp.where(idxm == N, idxm[:, 0:1], idxm)
    out_ref[0] = idxm


def _run_ballquery(new_xyz, ct):
    return pl.pallas_call(
        _bq_body,
        grid=(B, S // S_TILE),
        in_specs=[
            pl.BlockSpec((1, S_TILE, 3), lambda b, s: (b, s, 0)),
            pl.BlockSpec((1, 3, N), lambda b, s: (b, 0, 0)),
        ],
        out_specs=pl.BlockSpec((1, S_TILE, K), lambda b, s: (b, s, 0)),
        out_shape=jax.ShapeDtypeStruct((B, S, K), jnp.int32),
    )(new_xyz, ct)


TOTAL_ROWS = B * S * K + B * S
GCHUNK = 128


def _sc_gather(gidx2d, table):
    info = plsc.get_sparse_core_info()
    nw = info.num_cores * info.num_subcores
    rows_per_w = TOTAL_ROWS // nw
    chunks = rows_per_w // GCHUNK
    idx_rows = rows_per_w // GCHUNK
    mesh = plsc.VectorSubcoreMesh(core_axis_name="c", subcore_axis_name="s")

    @functools.partial(
        pl.kernel, mesh=mesh,
        compiler_params=pltpu.CompilerParams(use_tc_tiling_on_sc=False),
        out_type=jax.ShapeDtypeStruct((TOTAL_ROWS, CH), jnp.float32),
        scratch_types=[
            pltpu.VMEM((idx_rows, GCHUNK), jnp.int32),
            pltpu.VMEM((GCHUNK, CH), jnp.float32),
            pltpu.SemaphoreType.DMA,
        ],
    )
    def gk(idx_hbm, table_hbm, out_hbm, idx_v, rows_v, sem):
        wid = lax.axis_index("s") * info.num_cores + lax.axis_index("c")
        pltpu.sync_copy(idx_hbm.at[pl.ds(wid * idx_rows, idx_rows)], idx_v)
        base = wid * rows_per_w

        def chunk(c, carry):
            pltpu.async_copy(table_hbm.at[idx_v.at[c]], rows_v, sem).wait()
            pltpu.sync_copy(rows_v, out_hbm.at[pl.ds(base + c * GCHUNK, GCHUNK)])
            return carry

        lax.fori_loop(0, chunks, chunk, 0)

    return gk(gidx2d, table)


G_TILE = 64
GROWS = G_TILE * K
N_TILES = (B * S) // G_TILE


def _build_f(g_ref, np_ref):
    g3 = g_ref[...].reshape(G_TILE, K, CH)
    npb = np_ref[...]
    np3 = jnp.broadcast_to(npb[:, None, :], (G_TILE, K, CH))
    f3 = jnp.concatenate([g3 - np3, np3], axis=2)
    return f3.reshape(GROWS, 2 * CH)


_DN_RC = (((0,), (0,)), ((), ()))
_DN_CC = (((1,), (1,)), ((), ()))


def _eye64():
    r = lax.broadcasted_iota(jnp.int32, (C1, C1), 0)
    c = lax.broadcasted_iota(jnp.int32, (C1, C1), 1)
    return (r == c).astype(jnp.float32)


def _bn_scale_shift(W, m_mat, s_vec, gw, gb):
    mean_y = lax.dot_general(s_vec / MCOUNT, W, _DN_CC,
                             preferred_element_type=jnp.float32)
    A = lax.dot_general(W, m_mat, (((1,), (0,)), ((), ())),
                        preferred_element_type=jnp.float32)
    Km = lax.dot_general(A, W, _DN_CC, preferred_element_type=jnp.float32)
    e2 = jnp.sum(Km * _eye64(), axis=0, keepdims=True) / MCOUNT
    var = e2 - mean_y * mean_y
    scale = gw * lax.rsqrt(var + EPS)
    shift = gb - mean_y * scale
    return scale, shift


def _passA_body(g_ref, np_ref, m1_ref, s1_ref):
    f = _build_f(g_ref, np_ref)
    m1 = lax.dot_general(f, f, _DN_RC, preferred_element_type=jnp.float32)
    s1 = jnp.sum(f, axis=0, keepdims=True)

    @pl.when(pl.program_id(0) == 0)
    def _():
        m1_ref[...] = jnp.zeros_like(m1_ref)
        s1_ref[...] = jnp.zeros_like(s1_ref)

    m1_ref[...] += m1
    s1_ref[...] += s1


def _passB_body(g_ref, np_ref, w1_ref, m1_ref, s1_ref, g1w_ref, g1b_ref,
                m2_ref, s2_ref):
    f = _build_f(g_ref, np_ref)
    W1 = w1_ref[...]
    scale1, shift1 = _bn_scale_shift(W1, m1_ref[...], s1_ref[...],
                                     g1w_ref[...], g1b_ref[...])
    y1 = lax.dot_general(f, W1, _DN_CC, preferred_element_type=jnp.float32)
    z1 = jnp.maximum(y1 * scale1 + shift1, 0.0)
    m2 = lax.dot_general(z1, z1, _DN_RC, preferred_element_type=jnp.float32)
    s2 = jnp.sum(z1, axis=0, keepdims=True)

    @pl.when(pl.program_id(0) == 0)
    def _():
        m2_ref[...] = jnp.zeros_like(m2_ref)
        s2_ref[...] = jnp.zeros_like(s2_ref)

    m2_ref[...] += m2
    s2_ref[...] += s2


def _passC_body(g_ref, np_ref, w1_ref, m1_ref, s1_ref, g1w_ref, g1b_ref,
                w2_ref, m2_ref, s2_ref, g2w_ref, g2b_ref, out_ref):
    f = _build_f(g_ref, np_ref)
    W1 = w1_ref[...]
    scale1, shift1 = _bn_scale_shift(W1, m1_ref[...], s1_ref[...],
                                     g1w_ref[...], g1b_ref[...])
    y1 = lax.dot_general(f, W1, _DN_CC, preferred_element_type=jnp.float32)
    z1 = jnp.maximum(y1 * scale1 + shift1, 0.0)
    W2 = w2_ref[...]
    scale2, shift2 = _bn_scale_shift(W2, m2_ref[...], s2_ref[...],
                                     g2w_ref[...], g2b_ref[...])
    y2 = lax.dot_general(z1, W2, _DN_CC, preferred_element_type=jnp.float32)
    z2 = jnp.maximum(y2 * scale2 + shift2, 0.0)
    out_ref[...] = jnp.max(z2.reshape(G_TILE, K, C2), axis=1)


_SMALL = lambda shp: pl.BlockSpec(shp, lambda i: tuple(0 for _ in shp))


def _run_mlp(G, NP2, W1, W2, g1w, g1b, g2w, g2b):
    g_spec = pl.BlockSpec((GROWS, CH), lambda i: (i, 0))
    np_spec = pl.BlockSpec((G_TILE, CH), lambda i: (i, 0))
    acc_specs = (pl.BlockSpec((C1, C1), lambda i: (0, 0)),
                 pl.BlockSpec((1, C1), lambda i: (0, 0)))
    acc_shapes = (jax.ShapeDtypeStruct((C1, C1), jnp.float32),
                  jax.ShapeDtypeStruct((1, C1), jnp.float32))

    M1, s1 = pl.pallas_call(
        _passA_body, grid=(N_TILES,),
        in_specs=[g_spec, np_spec],
        out_specs=acc_specs, out_shape=acc_shapes,
    )(G, NP2)

    M2, s2 = pl.pallas_call(
        _passB_body, grid=(N_TILES,),
        in_specs=[g_spec, np_spec, _SMALL((C1, C1)), _SMALL((C1, C1)),
                  _SMALL((1, C1)), _SMALL((1, C1)), _SMALL((1, C1))],
        out_specs=acc_specs, out_shape=acc_shapes,
    )(G, NP2, W1, M1, s1, g1w, g1b)

    out2d = pl.pallas_call(
        _passC_body, grid=(N_TILES,),
        in_specs=[g_spec, np_spec, _SMALL((C1, C1)), _SMALL((C1, C1)),
                  _SMALL((1, C1)), _SMALL((1, C1)), _SMALL((1, C1)),
                  _SMALL((C2, C2)), _SMALL((C2, C2)), _SMALL((1, C2)),
                  _SMALL((1, C2)), _SMALL((1, C2))],
        out_specs=pl.BlockSpec((G_TILE, C2), lambda i: (i, 0)),
        out_shape=jax.ShapeDtypeStruct((B * S, C2), jnp.float32),
    )(G, NP2, W1, M1, s1, g1w, g1b, W2, M2, s2, g2w, g2b)
    return out2d


def kernel(x, coords, w1, w2, bn1_w, bn1_b, bn2_w, bn2_b):
    cx = coords[:, :, 0]
    cy = coords[:, :, 1]
    cz = coords[:, :, 2]
    ct = jnp.transpose(coords, (0, 2, 1))

    nxyz_tm, fidx_tm = _run_fps(cx, cy, cz)
    new_xyz = jnp.transpose(nxyz_tm, (1, 0, 2))
    fps_idx = jnp.transpose(fidx_tm[:, :, 0], (1, 0))
    idx = _run_ballquery(new_xyz, ct)

    boff = (jnp.arange(B, dtype=jnp.int32) * N)
    gidx = (idx + boff[:, None, None]).reshape(-1)
    cgidx = (fps_idx + boff[:, None]).reshape(-1)
    gidx2d = jnp.concatenate([gidx, cgidx]).reshape(TOTAL_ROWS // GCHUNK, GCHUNK)
    table = jnp.transpose(x, (0, 2, 1)).reshape(B * N, CH)
    Gall = _sc_gather(gidx2d, table)
    G = Gall[:B * S * K]
    NP2 = Gall[B * S * K:]

    out2d = _run_mlp(G, NP2, w1[:, :, 0], w2[:, :, 0],
                     bn1_w.reshape(1, C1), bn1_b.reshape(1, C1),
                     bn2_w.reshape(1, C2), bn2_b.reshape(1, C2))

    y = jnp.transpose(out2d.reshape(B, S, C2), (0, 2, 1))
    return (new_xyz, y)
